# baseline (speedup 1.0000x reference)
import functools

import jax
import jax.numpy as jnp
from jax import lax
from jax.experimental import pallas as pl
from jax.experimental.pallas import tpu as pltpu
from jax.experimental.pallas import tpu_sc as plsc

_BATCH = 16384
_DIM = 1024
_NC = 2
_NS = 16
_NW = _NC * _NS
_BPW = _BATCH // _NW
_CHUNK = 56
_NFULL = 9
_TAIL = _BPW - _NFULL * _CHUNK


def _make_sc_gather():
    mesh = plsc.VectorSubcoreMesh(core_axis_name="c", subcore_axis_name="s")

    @functools.partial(
        pl.kernel,
        mesh=mesh,
        out_type=jax.ShapeDtypeStruct((_BATCH, _DIM), jnp.float32),
        scratch_types=[
            pltpu.VMEM((_BPW,), jnp.int32),
            pltpu.VMEM((2, _CHUNK, _DIM), jnp.float32),
            pltpu.VMEM((_TAIL, _DIM), jnp.float32),
            pltpu.SemaphoreType.DMA,
            pltpu.SemaphoreType.DMA,
            pltpu.SemaphoreType.DMA,
            pltpu.SemaphoreType.DMA,
            pltpu.SemaphoreType.DMA,
        ],
    )
    def body(pos_hbm, table_hbm, out_hbm, idx_v, rows_v, tail_v,
             gsem0, gsem1, ssem0, ssem1, tsem):
        gsem = (gsem0, gsem1)
        ssem = (ssem0, ssem1)
        wid = lax.axis_index("s") * _NC + lax.axis_index("c")
        base = wid * _BPW
        pltpu.sync_copy(pos_hbm.at[pl.ds(base, _BPW)], idx_v)

        def start_gather(t, b):
            pltpu.async_copy(
                table_hbm.at[idx_v.at[pl.ds(t * _CHUNK, _CHUNK)]],
                rows_v.at[b], gsem[b])

        def wait_gather(b):
            pltpu.make_async_copy(
                table_hbm.at[idx_v.at[pl.ds(0, _CHUNK)]],
                rows_v.at[b], gsem[b]).wait()

        def start_store(t, b):
            pltpu.async_copy(
                rows_v.at[b],
                out_hbm.at[pl.ds(base + t * _CHUNK, _CHUNK)], ssem[b])

        def wait_store(b):
            pltpu.make_async_copy(
                rows_v.at[b], out_hbm.at[pl.ds(0, _CHUNK)], ssem[b]).wait()

        start_gather(0, 0)
        start_gather(1, 1)
        wait_gather(0)
        start_store(0, 0)

        def grp(m, carry):
            t0 = 2 * m
            wait_store(0)
            start_gather(t0, 0)
            wait_gather(1)
            start_store(t0 - 1, 1)
            wait_store(1)
            start_gather(t0 + 1, 1)
            wait_gather(0)
            start_store(t0, 0)
            return carry

        lax.fori_loop(1, 4, grp, 0)

        wait_store(0)
        start_gather(8, 0)
        tail = pltpu.async_copy(
            table_hbm.at[idx_v.at[pl.ds(_NFULL * _CHUNK, _TAIL)]],
            tail_v, tsem)
        wait_gather(1)
        start_store(7, 1)
        wait_gather(0)
        start_store(8, 0)
        tail.wait()
        pltpu.async_copy(
            tail_v, out_hbm.at[pl.ds(base + _NFULL * _CHUNK, _TAIL)],
            tsem)
        wait_store(1)
        wait_store(0)
        pltpu.make_async_copy(
            tail_v, out_hbm.at[pl.ds(0, _TAIL)], tsem).wait()

    return body


_sc_gather = _make_sc_gather()


@jax.jit
def kernel(pos, table):
    return _sc_gather(pos.astype(jnp.int32), table)


# speedup vs baseline: 1.0061x; 1.0061x over previous
import functools

import jax
import jax.numpy as jnp
from jax import lax
from jax.experimental import pallas as pl
from jax.experimental.pallas import tpu as pltpu
from jax.experimental.pallas import tpu_sc as plsc

_BATCH = 16384
_DIM = 1024
_NC = 2
_NS = 16
_NW = _NC * _NS
_BPW = _BATCH // _NW
_CHUNK = 32
_NCHUNK = _BPW // _CHUNK
_NBUF = 2
_NGRP = _NCHUNK // _NBUF


def _make_sc_gather():
    mesh = plsc.VectorSubcoreMesh(core_axis_name="c", subcore_axis_name="s")

    @functools.partial(
        pl.kernel,
        mesh=mesh,
        out_type=jax.ShapeDtypeStruct((_BATCH, _DIM), jnp.float32),
        scratch_types=[
            pltpu.VMEM((_BPW,), jnp.int32),
            pltpu.VMEM((_NBUF, _CHUNK, _DIM), jnp.float32),
            *([pltpu.SemaphoreType.DMA] * _NBUF),
            *([pltpu.SemaphoreType.DMA] * _NBUF),
        ],
    )
    def body(pos_hbm, table_hbm, out_hbm, idx_v, rows_v, *sems):
        gsem = sems[:_NBUF]
        ssem = sems[_NBUF:]
        wid = lax.axis_index("s") * _NC + lax.axis_index("c")
        base = wid * _BPW
        pltpu.sync_copy(pos_hbm.at[pl.ds(base, _BPW)], idx_v)

        def start_gather(t, b):
            pltpu.async_copy(
                table_hbm.at[idx_v.at[pl.ds(t * _CHUNK, _CHUNK)]],
                rows_v.at[b], gsem[b])

        def wait_gather(b):
            pltpu.make_async_copy(
                table_hbm.at[idx_v.at[pl.ds(0, _CHUNK)]],
                rows_v.at[b], gsem[b]).wait()

        def start_store(t, b):
            pltpu.async_copy(
                rows_v.at[b],
                out_hbm.at[pl.ds(base + t * _CHUNK, _CHUNK)], ssem[b])

        def wait_store(b):
            pltpu.make_async_copy(
                rows_v.at[b], out_hbm.at[pl.ds(0, _CHUNK)], ssem[b]).wait()

        start_gather(0, 0)
        start_gather(1, 1)
        wait_gather(0)
        start_store(0, 0)

        def grp(m, _):
            t0 = 2 * m
            wait_store(0)
            start_gather(t0, 0)
            wait_gather(1)
            start_store(t0 - 1, 1)
            wait_store(1)
            start_gather(t0 + 1, 1)
            wait_gather(0)
            start_store(t0, 0)
            return _

        lax.fori_loop(1, _NGRP, grp, 0)

        wait_gather(1)
        start_store(_NCHUNK - 1, 1)
        wait_store(0)
        wait_store(1)

    return body


_sc_gather = _make_sc_gather()


@jax.jit
def kernel(pos, table):
    return _sc_gather(pos.astype(jnp.int32), table)
